# Initial kernel scaffold; baseline (speedup 1.0000x reference)
import jax
import jax.numpy as jnp
from jax.experimental import pallas as pl


def kernel(lattice, fracs, species, batch_indices, num_atoms_list, emb, W1, b1, W2, b2, W3, b3, W4, b4):
    raise NotImplementedError("write your pallas kernel here")


# speedup vs baseline: 6.1760x; 6.1760x over previous
import jax
import jax.numpy as jnp
from jax import lax
from jax.experimental import pallas as pl
from jax.experimental.pallas import tpu as pltpu

N = 320000
B = 10000
BLK = 1600
NB = N // BLK
SPAN = 128
PBLK = 2000


def _f32dot(a, b):
    return jnp.dot(a, b, preferred_element_type=jnp.float32)


def _atom_kernel(lo_ref, hi_ref, species_ref, bi_ref, fracs_ref,
                 emb_ref, w1e_ref, w1f_ref, b1_ref, w2_ref, b2_ref,
                 pooled_ref, counts_ref, embw_ref):
    pid = pl.program_id(0)

    @pl.when(pid == 0)
    def _init():
        pooled_ref[...] = jnp.zeros_like(pooled_ref)
        counts_ref[...] = jnp.zeros_like(counts_ref)
        embw_ref[...] = _f32dot(emb_ref[...], w1e_ref[...])

    species = species_ref[0, 0, :]
    bi = bi_ref[0, 0, :]
    fr = fracs_ref[...]

    oh_sp = (species[:, None] ==
             lax.broadcasted_iota(jnp.int32, (BLK, 128), 1)).astype(jnp.float32)
    h1 = _f32dot(oh_sp, embw_ref[...]) + _f32dot(fr, w1f_ref[...]) + b1_ref[...]
    h1 = h1 * jax.nn.sigmoid(h1)
    h = _f32dot(h1, w2_ref[...]) + b2_ref[...]

    lo = lo_ref[pid]
    hi = hi_ref[pid]
    nsub = (hi - lo) // SPAN + 1

    def body(j, carry):
        win0 = lo + j * SPAN
        base = jnp.minimum(win0, B - SPAN)
        srows = lax.broadcasted_iota(jnp.int32, (SPAN, BLK), 0)
        inwin = (bi[None, :] >= win0) & (bi[None, :] < win0 + SPAN)
        ohT = ((bi[None, :] - base == srows) & inwin).astype(jnp.float32)
        contrib = _f32dot(ohT, h)
        cnt = jnp.sum(ohT, axis=1)
        pooled_ref[pl.ds(base, SPAN), :] += contrib
        counts_ref[pl.ds(base, SPAN), :] += jnp.broadcast_to(
            cnt[:, None], (SPAN, 128))
        return carry

    lax.fori_loop(0, nsub, body, 0)


def _pool_kernel(pooled_ref, counts_ref, lat_ref, w3p_ref, w3l_ref, b3_ref,
                 w4m_ref, w4l_ref, b4m_ref, b4l_ref, mu_ref, lv_ref):
    p = pooled_ref[...] / counts_ref[...]
    y = (_f32dot(p, w3p_ref[...]) + _f32dot(lat_ref[...], w3l_ref[...])
         + b3_ref[...])
    y = y * jax.nn.sigmoid(y)
    mu_ref[...] = _f32dot(y, w4m_ref[...]) + b4m_ref[...]
    lv_ref[...] = _f32dot(y, w4l_ref[...]) + b4l_ref[...]


def kernel(lattice, fracs, species, batch_indices, num_atoms_list,
           emb, W1, b1, W2, b2, W3, b3, W4, b4):
    species = species.astype(jnp.int32)
    bi = batch_indices.astype(jnp.int32)

    block_lo = bi[::BLK]
    block_hi = bi[BLK - 1::BLK]

    species3 = species.reshape(NB, 1, BLK)
    bi3 = bi.reshape(NB, 1, BLK)

    emb_p = jnp.pad(emb, ((0, 28), (0, 0)))
    w1e = W1[:, :64].T
    w1f = W1[:, 64:].T
    w2t = W2.T
    b1r = b1.reshape(1, 128)
    b2r = b2.reshape(1, 128)

    grid_spec = pltpu.PrefetchScalarGridSpec(
        num_scalar_prefetch=2,
        grid=(NB,),
        in_specs=[
            pl.BlockSpec((1, 1, BLK), lambda i, lo, hi: (i, 0, 0)),
            pl.BlockSpec((1, 1, BLK), lambda i, lo, hi: (i, 0, 0)),
            pl.BlockSpec((BLK, 3), lambda i, lo, hi: (i, 0)),
            pl.BlockSpec((128, 64), lambda i, lo, hi: (0, 0)),
            pl.BlockSpec((64, 128), lambda i, lo, hi: (0, 0)),
            pl.BlockSpec((3, 128), lambda i, lo, hi: (0, 0)),
            pl.BlockSpec((1, 128), lambda i, lo, hi: (0, 0)),
            pl.BlockSpec((128, 128), lambda i, lo, hi: (0, 0)),
            pl.BlockSpec((1, 128), lambda i, lo, hi: (0, 0)),
        ],
        out_specs=[
            pl.BlockSpec((B, 128), lambda i, lo, hi: (0, 0)),
            pl.BlockSpec((B, 128), lambda i, lo, hi: (0, 0)),
        ],
        scratch_shapes=[pltpu.VMEM((128, 128), jnp.float32)],
    )
    pooled, counts = pl.pallas_call(
        _atom_kernel,
        grid_spec=grid_spec,
        out_shape=[jax.ShapeDtypeStruct((B, 128), jnp.float32),
                   jax.ShapeDtypeStruct((B, 128), jnp.float32)],
    )(block_lo, block_hi, species3, bi3, fracs, emb_p, w1e, w1f, b1r, w2t, b2r)

    lat9 = lattice.reshape(B, 9)
    w3p = W3[:, :128].T
    w3l = W3[:, 128:].T
    w4m = W4[:128].T
    w4l = W4[128:].T
    b3r = b3.reshape(1, 128)
    b4m = b4[:128].reshape(1, 128)
    b4l = b4[128:].reshape(1, 128)

    mu, lv = pl.pallas_call(
        _pool_kernel,
        grid=(B // PBLK,),
        in_specs=[
            pl.BlockSpec((PBLK, 128), lambda i: (i, 0)),
            pl.BlockSpec((PBLK, 128), lambda i: (i, 0)),
            pl.BlockSpec((PBLK, 9), lambda i: (i, 0)),
            pl.BlockSpec((128, 128), lambda i: (0, 0)),
            pl.BlockSpec((9, 128), lambda i: (0, 0)),
            pl.BlockSpec((1, 128), lambda i: (0, 0)),
            pl.BlockSpec((128, 128), lambda i: (0, 0)),
            pl.BlockSpec((128, 128), lambda i: (0, 0)),
            pl.BlockSpec((1, 128), lambda i: (0, 0)),
            pl.BlockSpec((1, 128), lambda i: (0, 0)),
        ],
        out_specs=[
            pl.BlockSpec((PBLK, 128), lambda i: (i, 0)),
            pl.BlockSpec((PBLK, 128), lambda i: (i, 0)),
        ],
        out_shape=[jax.ShapeDtypeStruct((B, 128), jnp.float32),
                   jax.ShapeDtypeStruct((B, 128), jnp.float32)],
    )(pooled, counts, lat9, w3p, w3l, b3r, w4m, w4l, b4m, b4l)

    return (mu, lv)
